# baseline (speedup 1.0000x reference)
import functools

import jax
import jax.numpy as jnp
from jax import lax
from jax.experimental import pallas as pl
from jax.experimental.pallas import tpu as pltpu
from jax.experimental.pallas import tpu_sc as plsc

_D = 128
_T = 50
_TP = 56
_BB = 2


@functools.lru_cache(maxsize=None)
def _make_gather(batch, V):
    info = plsc.get_sparse_core_info()
    nw = info.num_cores * info.num_subcores
    assert batch % (nw * _BB) == 0
    b_per_w = batch // nw
    n_ch = b_per_w // _BB
    assert n_ch % 2 == 0 and n_ch >= 4
    mesh = plsc.VectorSubcoreMesh(core_axis_name="c", subcore_axis_name="s")

    @functools.partial(
        pl.kernel,
        mesh=mesh,
        out_type=jax.ShapeDtypeStruct((batch, _TP, _D), jnp.float32),
        scratch_types=[
            pltpu.VMEM((n_ch, _BB, _T), jnp.int32),
            pltpu.VMEM((2, _BB, _TP, _D), jnp.float32),
            pltpu.SemaphoreType.DMA((2,)),
            pltpu.SemaphoreType.DMA((2,)),
        ],
    )
    def gather_kernel(idx_hbm, table_hbm, out_hbm, idx_v, rows_v, gsem, osem):
        wid = lax.axis_index("s") * info.num_cores + lax.axis_index("c")
        bbase = wid * b_per_w
        pltpu.sync_copy(idx_hbm.at[wid], idx_v)

        def gather_start(g, b):
            for j in range(_BB):
                pltpu.async_copy(
                    table_hbm.at[idx_v.at[g, j]],
                    rows_v.at[b, j, pl.ds(0, _T)],
                    gsem.at[b],
                )

        def gather_wait(b):
            for j in range(_BB):
                pltpu.make_async_copy(
                    table_hbm.at[idx_v.at[0, j]],
                    rows_v.at[b, j, pl.ds(0, _T)],
                    gsem.at[b],
                ).wait()

        def out_start(g, b):
            pltpu.async_copy(
                rows_v.at[b],
                out_hbm.at[pl.ds(bbase + g * _BB, _BB)],
                osem.at[b],
            )

        def out_wait(b):
            pltpu.make_async_copy(
                rows_v.at[b],
                out_hbm.at[pl.ds(bbase, _BB)],
                osem.at[b],
            ).wait()

        gather_start(0, 0)
        gather_start(1, 1)
        gather_wait(0)
        out_start(0, 0)

        def body(t, carry):
            for i in range(2):
                g = 2 * t + 1 + i
                b = (1 + i) % 2
                nb = 1 - b
                out_wait(nb)
                gather_start(g + 1, nb)
                gather_wait(b)
                out_start(g, b)
            return carry

        lax.fori_loop(0, (n_ch - 2) // 2, body, 0)

        gather_wait(1)
        out_start(n_ch - 1, 1)
        out_wait(0)
        out_wait(1)

    return gather_kernel


_K = 4


def kernel(inputs, embedding):
    batch, steps = inputs.shape
    vocab, d = embedding.shape
    assert d == _D and steps == _T
    info = plsc.get_sparse_core_info()
    nw = info.num_cores * info.num_subcores
    shard = batch // _K
    n_ch = shard // (nw * _BB)
    idx = inputs.astype(jnp.int32).reshape(_K, nw, n_ch, _BB, _T)
    gk = _make_gather(shard, vocab)
    outs = [gk(idx[s], embedding)[:, :_T, :] for s in range(_K)]
    return jnp.concatenate(outs, axis=0)


# speedup vs baseline: 1.8520x; 1.8520x over previous
import functools

import jax
import jax.numpy as jnp
from jax import lax
from jax.experimental import pallas as pl
from jax.experimental.pallas import tpu as pltpu
from jax.experimental.pallas import tpu_sc as plsc

_D = 128
_T = 50
_BB = 2
_NB = 4


@functools.lru_cache(maxsize=None)
def _make_gather(batch, V):
    info = plsc.get_sparse_core_info()
    nw = info.num_cores * info.num_subcores
    assert batch % (nw * _BB) == 0
    b_per_w = batch // nw
    n_ch = b_per_w // _BB
    assert (n_ch - _NB) % _NB == 0 and n_ch >= 2 * _NB
    mesh = plsc.VectorSubcoreMesh(core_axis_name="c", subcore_axis_name="s")

    @functools.partial(
        pl.kernel,
        mesh=mesh,
        out_type=jax.ShapeDtypeStruct((batch, _T, _D), jnp.float32),
        scratch_types=[
            pltpu.VMEM((n_ch, _BB, _T), jnp.int32),
            pltpu.VMEM((_NB, _BB, _T, _D), jnp.float32),
            pltpu.SemaphoreType.DMA((_NB,)),
            pltpu.SemaphoreType.DMA((_NB,)),
        ],
    )
    def gather_kernel(idx_hbm, table_hbm, out_hbm, idx_v, rows_v, gsem, osem):
        wid = lax.axis_index("s") * info.num_cores + lax.axis_index("c")
        bbase = wid * b_per_w
        pltpu.sync_copy(idx_hbm.at[wid], idx_v)

        def gather_start(g, b):
            for j in range(_BB):
                pltpu.async_copy(
                    table_hbm.at[idx_v.at[g, j]],
                    rows_v.at[b, j],
                    gsem.at[b],
                )

        def gather_wait(b):
            for j in range(_BB):
                pltpu.make_async_copy(
                    table_hbm.at[idx_v.at[0, j]],
                    rows_v.at[b, j],
                    gsem.at[b],
                ).wait()

        def out_start(g, b):
            pltpu.async_copy(
                rows_v.at[b],
                out_hbm.at[pl.ds(bbase + g * _BB, _BB)],
                osem.at[b],
            )

        def out_wait(b):
            pltpu.make_async_copy(
                rows_v.at[b],
                out_hbm.at[pl.ds(bbase, _BB)],
                osem.at[b],
            ).wait()

        for g in range(_NB):
            gather_start(g, g)
        gather_wait(0)
        out_start(0, 0)

        def body(t, carry):
            for i in range(_NB):
                g = _NB * t + 1 + i
                b = (1 + i) % _NB
                out_wait((b - 1) % _NB)
                gather_start(g + _NB - 1, (b - 1) % _NB)
                gather_wait(b)
                out_start(g, b)
            return carry

        lax.fori_loop(0, (n_ch - _NB) // _NB, body, 0)

        for g in range(n_ch - _NB + 1, n_ch):
            b = g % _NB
            out_wait((b - 1) % _NB)
            gather_wait(b)
            out_start(g, b)
        out_wait((n_ch - 1) % _NB)

    return gather_kernel


def kernel(inputs, embedding):
    batch, steps = inputs.shape
    vocab, d = embedding.shape
    assert d == _D and steps == _T
    info = plsc.get_sparse_core_info()
    nw = info.num_cores * info.num_subcores
    n_ch = batch // (nw * _BB)
    idx = inputs.astype(jnp.int32).reshape(nw, n_ch, _BB, _T)
    return _make_gather(batch, vocab)(idx, embedding)
